# Initial kernel scaffold; baseline (speedup 1.0000x reference)
import jax
import jax.numpy as jnp
from jax.experimental import pallas as pl


def kernel(indices, tables):
    raise NotImplementedError("write your pallas kernel here")


# speedup vs baseline: 24.5732x; 24.5732x over previous
import functools

import jax
import jax.numpy as jnp
from jax import lax
from jax.experimental import pallas as pl
from jax.experimental.pallas import tpu as pltpu
from jax.experimental.pallas import tpu_sc as plsc

_NC, _NS, _L = 2, 16, 16
_NW = _NC * _NS

_IDX_W = 128


@functools.lru_cache(maxsize=None)
def _build(B, F, V, D):
    rows_per_chunk = 64
    idxc = rows_per_chunk * F
    n_idx_rows = idxc // _IDX_W
    b_per_w = B // _NW
    chunks = b_per_w // rows_per_chunk
    assert idxc % _IDX_W == 0 and b_per_w % rows_per_chunk == 0
    assert _IDX_W % _L == 0

    w_idx_rows = chunks * n_idx_rows
    assert w_idx_rows % 8 == 0

    mesh = plsc.VectorSubcoreMesh(core_axis_name="c", subcore_axis_name="s")

    @functools.partial(
        pl.kernel,
        mesh=mesh,
        compiler_params=pltpu.CompilerParams(use_tc_tiling_on_sc=False),
        out_type=jax.ShapeDtypeStruct((B * F, D), jnp.float32),
        scratch_types=[
            pltpu.VMEM((w_idx_rows, _IDX_W), jnp.int32),
            pltpu.VMEM((w_idx_rows, _IDX_W), jnp.int32),
            pltpu.VMEM((idxc, D), jnp.float32),
            pltpu.SemaphoreType.DMA,
        ],
    )
    def tok(idx_hbm, off_hbm, tab_hbm, out_hbm, idx_v, off_v, rows_v, sem):
        wid = lax.axis_index("s") * _NC + lax.axis_index("c")
        pltpu.sync_copy(off_hbm, off_v)
        pltpu.sync_copy(idx_hbm.at[pl.ds(wid * w_idx_rows, w_idx_rows)], idx_v)
        base_flat = wid * (chunks * idxc)

        def chunk_body(c, carry):
            for j in range(n_idx_rows):
                r = c * n_idx_rows + j
                for k in range(_IDX_W // _L):
                    s = pl.ds(k * _L, _L)
                    idx_v[r, s] = idx_v[r, s] + off_v[r, s]
            copies = [
                pltpu.async_copy(
                    tab_hbm.at[idx_v.at[c * n_idx_rows + j]],
                    rows_v.at[pl.ds(j * _IDX_W, _IDX_W)],
                    sem,
                )
                for j in range(n_idx_rows)
            ]
            for cp in copies:
                cp.wait()
            pltpu.sync_copy(
                rows_v, out_hbm.at[pl.ds(base_flat + c * idxc, idxc)]
            )
            return carry

        lax.fori_loop(0, chunks, chunk_body, 0)

    off = ((jnp.arange(w_idx_rows * _IDX_W, dtype=jnp.int32) % F) * V).reshape(
        w_idx_rows, _IDX_W)
    return tok, off


def kernel(indices, tables):
    B, F = indices.shape
    F2, V, D = tables.shape
    assert F2 == F
    tok, off = _build(B, F, V, D)
    idx2 = indices.astype(jnp.int32).reshape((B * F) // _IDX_W, _IDX_W)
    tab = tables.reshape(F * V, D)
    out = tok(idx2, off, tab)
    return out.reshape(B, F * D)
